# baseline (speedup 1.0000x reference)
import jax
import jax.numpy as jnp
from jax import lax
from jax.experimental import pallas as pl
from jax.experimental.pallas import tpu as pltpu

H = 128
GROUPS = 16
PER_GROUP = 8
SLOTS = GROUPS * PER_GROUP
B = 4096
ROW = SLOTS * H
OUT = GROUPS * H

BR = 16
NBLK = B // BR
NB = 4


def _body(x_hbm, o_ref, bufs, acc, sems):
    def _start(t, b):
        pltpu.make_async_copy(
            x_hbm.at[pl.ds(t * BR, BR), :], bufs.at[b], sems[b]
        ).start()

    def _wait(b):
        pltpu.make_async_copy(
            x_hbm.at[pl.ds(0, BR), :], bufs.at[b], sems[b]
        ).wait()

    def _fold(b):
        return jnp.max(bufs[b].reshape(BR // 8, 8, ROW), axis=0)

    for b in range(NB):
        _start(b, b)

    _wait(0)
    acc[...] = _fold(0)
    _start(NB, 0)
    for b in range(1, NB):
        _wait(b)
        acc[...] = jnp.maximum(acc[...], _fold(b))
        _start(NB + b, b)

    def _loop(k, _):
        t = NB + k * NB
        for b in range(NB):
            _wait(b)
            acc[...] = jnp.maximum(acc[...], _fold(b))

            @pl.when(t + NB + b < NBLK)
            def _next():
                _start(t + NB + b, b)

        return 0

    lax.fori_loop(0, NBLK // NB - 1, _loop, 0)

    a = jnp.max(acc[...], axis=0)
    a = jnp.max(a.reshape(PER_GROUP, GROUPS, H), axis=0)
    o_ref[...] = a.reshape(1, OUT)


def kernel(x):
    return pl.pallas_call(
        _body,
        in_specs=[pl.BlockSpec(memory_space=pl.ANY)],
        out_specs=pl.BlockSpec(memory_space=pltpu.MemorySpace.VMEM),
        out_shape=jax.ShapeDtypeStruct((1, OUT), jnp.float32),
        scratch_shapes=[
            pltpu.VMEM((NB, BR, ROW), jnp.float32),
            pltpu.VMEM((8, ROW), jnp.float32),
            [pltpu.SemaphoreType.DMA] * NB,
        ],
    )(x)


# speedup vs baseline: 1.2091x; 1.2091x over previous
import jax
import jax.numpy as jnp
from jax import lax
from jax.experimental import pallas as pl
from jax.experimental.pallas import tpu as pltpu

H = 128
GROUPS = 16
PER_GROUP = 8
SLOTS = GROUPS * PER_GROUP
B = 4096
ROW = SLOTS * H
OUT = GROUPS * H

BR = 32
NBLK = B // BR
NB = 8


def _body(x_hbm, o_ref, bufs, acc, sems):
    def _start(t, b):
        pltpu.make_async_copy(
            x_hbm.at[pl.ds(t * BR, BR), :], bufs.at[b], sems[b]
        ).start()

    def _wait(b):
        pltpu.make_async_copy(
            x_hbm.at[pl.ds(0, BR), :], bufs.at[b], sems[b]
        ).wait()

    def _fold(b):
        return jnp.max(bufs[b].reshape(BR // 8, 8, ROW), axis=0)

    for b in range(NB):
        _start(b, b)

    _wait(0)
    acc[...] = _fold(0)
    _start(NB, 0)
    for b in range(1, NB):
        _wait(b)
        acc[...] = jnp.maximum(acc[...], _fold(b))
        _start(NB + b, b)

    def _loop(k, _):
        t = NB + k * NB
        for b in range(NB):
            _wait(b)
            acc[...] = jnp.maximum(acc[...], _fold(b))

            @pl.when(t + NB + b < NBLK)
            def _next():
                _start(t + NB + b, b)

        return 0

    lax.fori_loop(0, NBLK // NB - 1, _loop, 0)

    a = jnp.max(acc[...], axis=0)
    a = jnp.max(a.reshape(PER_GROUP, GROUPS, H), axis=0)
    o_ref[...] = a.reshape(1, OUT)


def kernel(x):
    return pl.pallas_call(
        _body,
        in_specs=[pl.BlockSpec(memory_space=pl.ANY)],
        out_specs=pl.BlockSpec(memory_space=pltpu.MemorySpace.VMEM),
        out_shape=jax.ShapeDtypeStruct((1, OUT), jnp.float32),
        scratch_shapes=[
            pltpu.VMEM((NB, BR, ROW), jnp.float32),
            pltpu.VMEM((8, ROW), jnp.float32),
            [pltpu.SemaphoreType.DMA] * NB,
        ],
    )(x)


# speedup vs baseline: 1.2132x; 1.0034x over previous
import jax
import jax.numpy as jnp
from jax import lax
from jax.experimental import pallas as pl
from jax.experimental.pallas import tpu as pltpu

H = 128
GROUPS = 16
PER_GROUP = 8
SLOTS = GROUPS * PER_GROUP
B = 4096
ROW = SLOTS * H
OUT = GROUPS * H

BR = 32
NBLK = B // BR
NB = 4


def _body(x_hbm, o_ref, bufs, acc, sems):
    def _start(t, b):
        pltpu.make_async_copy(
            x_hbm.at[pl.ds(t * BR, BR), :], bufs.at[b], sems[b]
        ).start()

    def _wait(b):
        pltpu.make_async_copy(
            x_hbm.at[pl.ds(0, BR), :], bufs.at[b], sems[b]
        ).wait()

    def _fold(b):
        return jnp.max(bufs[b].reshape(BR // 8, 8, ROW), axis=0)

    for b in range(NB):
        _start(b, b)

    _wait(0)
    acc[...] = _fold(0)
    _start(NB, 0)
    for b in range(1, NB):
        _wait(b)
        acc[...] = jnp.maximum(acc[...], _fold(b))
        _start(NB + b, b)

    def _loop(k, _):
        t = NB + k * NB
        for b in range(NB):
            _wait(b)
            acc[...] = jnp.maximum(acc[...], _fold(b))

            @pl.when(t + NB + b < NBLK)
            def _next():
                _start(t + NB + b, b)

        return 0

    lax.fori_loop(0, NBLK // NB - 1, _loop, 0)

    a = jnp.max(acc[...], axis=0)
    a = jnp.max(a.reshape(PER_GROUP, GROUPS, H), axis=0)
    o_ref[...] = a.reshape(1, OUT)


def kernel(x):
    return pl.pallas_call(
        _body,
        in_specs=[pl.BlockSpec(memory_space=pl.ANY)],
        out_specs=pl.BlockSpec(memory_space=pltpu.MemorySpace.VMEM),
        out_shape=jax.ShapeDtypeStruct((1, OUT), jnp.float32),
        scratch_shapes=[
            pltpu.VMEM((NB, BR, ROW), jnp.float32),
            pltpu.VMEM((8, ROW), jnp.float32),
            [pltpu.SemaphoreType.DMA] * NB,
        ],
    )(x)
